# Initial kernel scaffold; baseline (speedup 1.0000x reference)
import jax
import jax.numpy as jnp
from jax.experimental import pallas as pl


def kernel(confidences, correct):
    raise NotImplementedError("write your pallas kernel here")


# speedup vs baseline: 1.2019x; 1.2019x over previous
import jax
import jax.numpy as jnp
from jax import lax
from jax.experimental import pallas as pl
from jax.experimental.pallas import tpu as pltpu
from jax.experimental.pallas import tpu_sc as plsc

N = 1000000
NUM_BINS = 10
L = 16
NC = 2
NS = 16
NW = NC * NS
VREGS = N // L
VPW = VREGS // NW
REM = VREGS - VPW * NW
CHUNK = VPW * L
PART = NUM_BINS * L
PADPART = 256
ROW = 2 * PADPART


def _bin_and_scatter(cf_part, ac_part, conf, corr, lane):
    corrf = corr.astype(jnp.float32)
    b = (conf * jnp.float32(NUM_BINS)).astype(jnp.int32)
    b = jnp.minimum(jnp.maximum(b, 0), NUM_BINS - 1)
    lin = b * L + lane
    plsc.addupdate_scatter(cf_part, [lin], conf)
    plsc.addupdate_scatter(ac_part, [lin], corrf)


def _partials_body(conf_hbm, corr_hbm, out_hbm,
                   conf_v, corr_v, tailc_v, tailr_v, cf_part, ac_part):
    gid = lax.axis_index("s") * NC + lax.axis_index("c")
    lane = lax.iota(jnp.int32, L)

    for r in range(PADPART // L):
        cf_part[pl.ds(r * L, L)] = jnp.zeros((L,), jnp.float32)
        ac_part[pl.ds(r * L, L)] = jnp.zeros((L,), jnp.float32)

    base = gid * CHUNK
    pltpu.sync_copy(conf_hbm.at[pl.ds(base, CHUNK)], conf_v)
    pltpu.sync_copy(corr_hbm.at[pl.ds(base, CHUNK)], corr_v)

    def it(i, carry):
        off = i * L
        conf = conf_v[pl.ds(off, L)]
        corr = corr_v[pl.ds(off, L)]
        _bin_and_scatter(cf_part, ac_part, conf, corr, lane)
        return carry

    lax.fori_loop(0, VPW, it, 0)

    @pl.when(gid < REM)
    def _tail():
        tbase = NW * CHUNK + gid * L
        pltpu.sync_copy(conf_hbm.at[pl.ds(tbase, L)], tailc_v)
        pltpu.sync_copy(corr_hbm.at[pl.ds(tbase, L)], tailr_v)
        _bin_and_scatter(cf_part, ac_part, tailc_v[...], tailr_v[...], lane)

    pltpu.sync_copy(cf_part, out_hbm.at[gid, 0, pl.ds(0, PADPART)])
    pltpu.sync_copy(ac_part, out_hbm.at[gid, 0, pl.ds(PADPART, PADPART)])


_partials_sc = pl.kernel(
    _partials_body,
    out_type=jax.ShapeDtypeStruct((NW, 1, ROW), jnp.float32),
    mesh=plsc.VectorSubcoreMesh(
        core_axis_name="c", subcore_axis_name="s", num_cores=NC),
    compiler_params=pltpu.CompilerParams(needs_layout_passes=False),
    scratch_types=[
        pltpu.VMEM((CHUNK,), jnp.float32),
        pltpu.VMEM((CHUNK,), jnp.int32),
        pltpu.VMEM((L,), jnp.float32),
        pltpu.VMEM((L,), jnp.int32),
        pltpu.VMEM((PADPART,), jnp.float32),
        pltpu.VMEM((PADPART,), jnp.float32),
    ],
)


def _finalize_body(p_ref, o_ref):
    x = p_ref[...][:, 0, :]
    diff = x[:, :PART] - x[:, PADPART:PADPART + PART]
    ece = jnp.float32(0.0)
    for b in range(NUM_BINS):
        ece = ece + jnp.abs(jnp.sum(diff[:, b * L:(b + 1) * L]))
    o_ref[...] = jnp.full((1, 1), ece * jnp.float32(1.0 / N))


_finalize_tc = pl.pallas_call(
    _finalize_body,
    out_shape=jax.ShapeDtypeStruct((1, 1), jnp.float32),
)


def kernel(confidences, correct):
    partials = _partials_sc(confidences, correct)
    return _finalize_tc(partials)[0, 0]


# speedup vs baseline: 1.8778x; 1.5623x over previous
import jax
import jax.numpy as jnp
from jax import lax
from jax.experimental import pallas as pl
from jax.experimental.pallas import tpu as pltpu
from jax.experimental.pallas import tpu_sc as plsc

N = 1000000
NUM_BINS = 10
L = 16
NC = 2
NS = 16
NW = NC * NS
VREGS = N // L
VPW = VREGS // NW
REM = VREGS - VPW * NW
CHUNK = VPW * L
PART = NUM_BINS * L
PADPART = 256
ROW = 2 * PADPART


def _bin_and_scatter(cf_part, ac_part, conf, corr, lane):
    corrf = corr.astype(jnp.float32)
    b = (conf * jnp.float32(NUM_BINS)).astype(jnp.int32)
    b = jnp.minimum(jnp.maximum(b, 0), NUM_BINS - 1)
    lin = b * L + lane
    plsc.addupdate_scatter(cf_part, [lin], conf)
    plsc.addupdate_scatter(ac_part, [lin], corrf)


def _partials_body(conf_hbm, corr_hbm, out_hbm,
                   conf_v, corr_v, tailc_v, tailr_v, cf_part, ac_part):
    gid = lax.axis_index("s") * NC + lax.axis_index("c")
    lane = lax.iota(jnp.int32, L)

    for r in range(PADPART // L):
        cf_part[pl.ds(r * L, L)] = jnp.zeros((L,), jnp.float32)
        ac_part[pl.ds(r * L, L)] = jnp.zeros((L,), jnp.float32)

    base = gid * CHUNK
    pltpu.sync_copy(conf_hbm.at[pl.ds(base, CHUNK)], conf_v)
    pltpu.sync_copy(corr_hbm.at[pl.ds(base, CHUNK)], corr_v)

    @plsc.parallel_loop(0, VPW, 1, unroll=16)
    def _it(i):
        off = i * L
        conf = conf_v[pl.ds(off, L)]
        corr = corr_v[pl.ds(off, L)]
        _bin_and_scatter(cf_part, ac_part, conf, corr, lane)

    @pl.when(gid < REM)
    def _tail():
        tbase = NW * CHUNK + gid * L
        pltpu.sync_copy(conf_hbm.at[pl.ds(tbase, L)], tailc_v)
        pltpu.sync_copy(corr_hbm.at[pl.ds(tbase, L)], tailr_v)
        _bin_and_scatter(cf_part, ac_part, tailc_v[...], tailr_v[...], lane)

    pltpu.sync_copy(cf_part, out_hbm.at[gid, 0, pl.ds(0, PADPART)])
    pltpu.sync_copy(ac_part, out_hbm.at[gid, 0, pl.ds(PADPART, PADPART)])


_partials_sc = pl.kernel(
    _partials_body,
    out_type=jax.ShapeDtypeStruct((NW, 1, ROW), jnp.float32),
    mesh=plsc.VectorSubcoreMesh(
        core_axis_name="c", subcore_axis_name="s", num_cores=NC),
    compiler_params=pltpu.CompilerParams(needs_layout_passes=False),
    scratch_types=[
        pltpu.VMEM((CHUNK,), jnp.float32),
        pltpu.VMEM((CHUNK,), jnp.int32),
        pltpu.VMEM((L,), jnp.float32),
        pltpu.VMEM((L,), jnp.int32),
        pltpu.VMEM((PADPART,), jnp.float32),
        pltpu.VMEM((PADPART,), jnp.float32),
    ],
)


def _finalize_body(p_ref, o_ref):
    x = p_ref[...][:, 0, :]
    diff = x[:, :PART] - x[:, PADPART:PADPART + PART]
    ece = jnp.float32(0.0)
    for b in range(NUM_BINS):
        ece = ece + jnp.abs(jnp.sum(diff[:, b * L:(b + 1) * L]))
    o_ref[...] = jnp.full((1, 1), ece * jnp.float32(1.0 / N))


_finalize_tc = pl.pallas_call(
    _finalize_body,
    out_shape=jax.ShapeDtypeStruct((1, 1), jnp.float32),
)


def kernel(confidences, correct):
    partials = _partials_sc(confidences, correct)
    return _finalize_tc(partials)[0, 0]


# speedup vs baseline: 2.0399x; 1.0863x over previous
import jax
import jax.numpy as jnp
from jax import lax
from jax.experimental import pallas as pl
from jax.experimental.pallas import tpu as pltpu
from jax.experimental.pallas import tpu_sc as plsc

N = 1000000
NUM_BINS = 10
L = 16
NC = 2
NS = 16
NW = NC * NS
VREGS = N // L
VPW = VREGS // NW
REM = VREGS - VPW * NW
CHUNK = VPW * L
NSUB = 3
SUBV = VPW // NSUB
SUB = SUBV * L
PART = NUM_BINS * L
PADPART = 256
ROW = 2 * PADPART


def _bin_and_scatter(cf_part, ac_part, conf, corr, lane):
    corrf = corr.astype(jnp.float32)
    b = (conf * jnp.float32(NUM_BINS)).astype(jnp.int32)
    b = jnp.minimum(b, NUM_BINS - 1)
    lin = b * L + lane
    plsc.addupdate_scatter(cf_part, [lin], conf)
    plsc.addupdate_scatter(ac_part, [lin], corrf)


def _partials_body(conf_hbm, corr_hbm, out_hbm,
                   conf_v0, conf_v1, corr_v0, corr_v1,
                   tailc_v, tailr_v, cf_part, ac_part,
                   sem_c0, sem_r0, sem_c1, sem_r1):
    gid = lax.axis_index("s") * NC + lax.axis_index("c")
    lane = lax.iota(jnp.int32, L)

    for r in range(PADPART // L):
        cf_part[pl.ds(r * L, L)] = jnp.zeros((L,), jnp.float32)
        ac_part[pl.ds(r * L, L)] = jnp.zeros((L,), jnp.float32)

    base = gid * CHUNK
    slots = [(conf_v0, corr_v0, sem_c0, sem_r0),
             (conf_v1, corr_v1, sem_c1, sem_r1)]

    def start(k):
        cv, rv, sc, sr = slots[k % 2]
        off = base + k * SUB
        hc = pltpu.async_copy(conf_hbm.at[pl.ds(off, SUB)], cv, sc)
        hr = pltpu.async_copy(corr_hbm.at[pl.ds(off, SUB)], rv, sr)
        return hc, hr

    pending = start(0)
    for k in range(NSUB):
        cv, rv, _, _ = slots[k % 2]
        hc, hr = pending
        hc.wait()
        hr.wait()
        if k + 1 < NSUB:
            pending = start(k + 1)

        @plsc.parallel_loop(0, SUBV, 1, unroll=16)
        def _it(i):
            off = i * L
            conf = cv[pl.ds(off, L)]
            corr = rv[pl.ds(off, L)]
            _bin_and_scatter(cf_part, ac_part, conf, corr, lane)

    @pl.when(gid < REM)
    def _tail():
        tbase = NW * CHUNK + gid * L
        pltpu.sync_copy(conf_hbm.at[pl.ds(tbase, L)], tailc_v)
        pltpu.sync_copy(corr_hbm.at[pl.ds(tbase, L)], tailr_v)
        _bin_and_scatter(cf_part, ac_part, tailc_v[...], tailr_v[...], lane)

    pltpu.sync_copy(cf_part, out_hbm.at[gid, 0, pl.ds(0, PADPART)])
    pltpu.sync_copy(ac_part, out_hbm.at[gid, 0, pl.ds(PADPART, PADPART)])


_partials_sc = pl.kernel(
    _partials_body,
    out_type=jax.ShapeDtypeStruct((NW, 1, ROW), jnp.float32),
    mesh=plsc.VectorSubcoreMesh(
        core_axis_name="c", subcore_axis_name="s", num_cores=NC),
    compiler_params=pltpu.CompilerParams(needs_layout_passes=False),
    scratch_types=[
        pltpu.VMEM((SUB,), jnp.float32),
        pltpu.VMEM((SUB,), jnp.float32),
        pltpu.VMEM((SUB,), jnp.int32),
        pltpu.VMEM((SUB,), jnp.int32),
        pltpu.VMEM((L,), jnp.float32),
        pltpu.VMEM((L,), jnp.int32),
        pltpu.VMEM((PADPART,), jnp.float32),
        pltpu.VMEM((PADPART,), jnp.float32),
        pltpu.SemaphoreType.DMA,
        pltpu.SemaphoreType.DMA,
        pltpu.SemaphoreType.DMA,
        pltpu.SemaphoreType.DMA,
    ],
)


def _finalize_body(p_ref, o_ref):
    x = p_ref[...][:, 0, :]
    diff = x[:, :PART] - x[:, PADPART:PADPART + PART]
    ece = jnp.float32(0.0)
    for b in range(NUM_BINS):
        ece = ece + jnp.abs(jnp.sum(diff[:, b * L:(b + 1) * L]))
    o_ref[...] = jnp.full((1, 1), ece * jnp.float32(1.0 / N))


_finalize_tc = pl.pallas_call(
    _finalize_body,
    out_shape=jax.ShapeDtypeStruct((1, 1), jnp.float32),
)


def kernel(confidences, correct):
    partials = _partials_sc(confidences, correct)
    return _finalize_tc(partials)[0, 0]
